# baseline (speedup 1.0000x reference)
import functools

import jax
import jax.numpy as jnp
from jax import lax
from jax.experimental import pallas as pl
from jax.experimental.pallas import tpu as pltpu
from jax.experimental.pallas import tpu_sc as plsc

_NC = 2
_NS = 16
_NW = _NC * _NS
_CH = 32
_NBUF = 3


def _make_sc_copy(seq, d, dtype):
    rows_per_w = seq // _NW
    nchunk = rows_per_w // _CH
    mesh = plsc.VectorSubcoreMesh(core_axis_name="c", subcore_axis_name="s")

    @functools.partial(
        pl.kernel,
        out_type=jax.ShapeDtypeStruct((seq, d), dtype),
        mesh=mesh,
        scratch_types=[
            pltpu.VMEM_SHARED((_NS, _NBUF, _CH, d), dtype),
            pltpu.SemaphoreType.DMA((_NBUF,)),
            pltpu.SemaphoreType.DMA((_NBUF,)),
        ],
    )
    def sc_copy(table_hbm, out_hbm, buf, isem, osem):
        cid = lax.axis_index("c")
        sid = lax.axis_index("s")
        wid = sid * _NC + cid
        base = wid * rows_per_w

        def in_cp(i):
            s = i % _NBUF
            return pltpu.make_async_copy(
                table_hbm.at[pl.ds(base + i * _CH, _CH)],
                buf.at[sid, s], isem.at[s])

        def out_cp(i):
            s = i % _NBUF
            return pltpu.make_async_copy(
                buf.at[sid, s],
                out_hbm.at[pl.ds(base + i * _CH, _CH)], osem.at[s])

        waited = set()
        for i in range(min(_NBUF - 1, nchunk)):
            in_cp(i).start()
        for i in range(nchunk):
            in_cp(i).wait()
            out_cp(i).start()
            nxt = i + _NBUF - 1
            if nxt < nchunk:
                if i >= 1:
                    out_cp(i - 1).wait()
                    waited.add(i - 1)
                in_cp(nxt).start()
        for j in range(nchunk):
            if j not in waited:
                out_cp(j).wait()

    return sc_copy


def kernel(x, positional_embeddings):
    seq = x.shape[1]
    table = positional_embeddings
    src = table if seq == table.shape[0] else table[:seq]
    return _make_sc_copy(seq, table.shape[1], table.dtype)(src)


# speedup vs baseline: 1.0286x; 1.0286x over previous
import functools

import jax
import jax.numpy as jnp
from jax import lax
from jax.experimental import pallas as pl
from jax.experimental.pallas import tpu as pltpu
from jax.experimental.pallas import tpu_sc as plsc

_NC = 2
_NS = 16
_NW = _NC * _NS
_CH = 16
_NBUF = 4


def _make_sc_copy(seq, d, dtype):
    rows_per_w = seq // _NW
    nchunk = rows_per_w // _CH
    mesh = plsc.VectorSubcoreMesh(core_axis_name="c", subcore_axis_name="s")

    @functools.partial(
        pl.kernel,
        out_type=jax.ShapeDtypeStruct((seq, d), dtype),
        mesh=mesh,
        scratch_types=[
            pltpu.VMEM((_NBUF, _CH, d), dtype),
            pltpu.SemaphoreType.DMA((_NBUF,)),
            pltpu.SemaphoreType.DMA((_NBUF,)),
        ],
    )
    def sc_copy(table_hbm, out_hbm, buf, isem, osem):
        wid = lax.axis_index("s") * _NC + lax.axis_index("c")
        base = wid * rows_per_w

        def in_cp(i):
            s = lax.rem(i, _NBUF)
            return pltpu.make_async_copy(
                table_hbm.at[pl.ds(base + i * _CH, _CH)], buf.at[s], isem.at[s])

        def out_cp(i):
            s = lax.rem(i, _NBUF)
            return pltpu.make_async_copy(
                buf.at[s], out_hbm.at[pl.ds(base + i * _CH, _CH)], osem.at[s])

        for i in range(_NBUF - 1):
            in_cp(jnp.int32(i)).start()

        def body(i, carry):
            in_cp(i).wait()
            out_cp(i).start()

            @pl.when(i >= 1)
            def _():
                out_cp(i - 1).wait()

            @pl.when(i + _NBUF - 1 < nchunk)
            def _():
                in_cp(i + _NBUF - 1).start()

            return carry

        lax.fori_loop(0, nchunk, body, jnp.int32(0))
        out_cp(jnp.int32(nchunk - 1)).wait()

    return sc_copy


def kernel(x, positional_embeddings):
    seq = x.shape[1]
    table = positional_embeddings
    src = table if seq == table.shape[0] else table[:seq]
    return _make_sc_copy(seq, table.shape[1], table.dtype)(src)
